# baseline (speedup 1.0000x reference)
import jax
import jax.numpy as jnp
from jax import lax
from jax.experimental import pallas as pl
from jax.experimental.pallas import tpu as pltpu
from jax.experimental.pallas import tpu_sc as plsc

TOKENS = 4096 * 200
D = 128
NC, NS = 2, 16
NW = NC * NS
B_PER_W = TOKENS // NW
C = 128
GPB = 2
ROWS = C * GPB
NSTEPS = B_PER_W // ROWS
NBUF = 2
NSUPER = NSTEPS // NBUF


def _emb_body(x_hbm, table_hbm, out_hbm, idx_v, bufs, gsem, ssem):
    wid = lax.axis_index("s") * NC + lax.axis_index("c")
    base = wid * B_PER_W

    pltpu.sync_copy(x_hbm.at[wid], idx_v)

    def fire_gather(b, c):
        for j in range(GPB):
            pltpu.async_copy(
                table_hbm.at[idx_v.at[c * GPB + j]],
                bufs.at[b, pl.ds(j * C, C)],
                gsem.at[b],
            )

    def wait_gather(b):
        pltpu.make_async_copy(
            table_hbm.at[pl.ds(0, ROWS)], bufs.at[b], gsem.at[b]
        ).wait()

    def fire_store(b, c):
        pltpu.async_copy(
            bufs.at[b], out_hbm.at[pl.ds(base + c * ROWS, ROWS)], ssem.at[b]
        )

    def wait_store(b):
        pltpu.make_async_copy(
            bufs.at[b], out_hbm.at[pl.ds(0, ROWS)], ssem.at[b]
        ).wait()

    for b in range(NBUF):
        fire_gather(b, b)

    def body(s, carry):
        g = s * NBUF
        for b in range(NBUF):
            wait_gather(b)
            fire_store(b, g + b)
        for b in range(NBUF):
            wait_store(b)
            fire_gather(b, g + b + NBUF)
        return carry

    lax.fori_loop(0, NSUPER - 1, body, 0, unroll=False)

    g = (NSUPER - 1) * NBUF
    for b in range(NBUF):
        wait_gather(b)
        fire_store(b, g + b)
    for b in range(NBUF):
        wait_store(b)


@jax.jit
def _emb(x3, table):
    mesh = plsc.VectorSubcoreMesh(
        core_axis_name="c", subcore_axis_name="s", num_cores=NC, num_subcores=NS
    )
    return pl.kernel(
        _emb_body,
        out_type=jax.ShapeDtypeStruct((TOKENS, D), jnp.float32),
        mesh=mesh,
        scratch_types=[
            pltpu.VMEM((NSTEPS * GPB, C), jnp.int32),
            pltpu.VMEM((NBUF, ROWS, D), jnp.float32),
            pltpu.SemaphoreType.DMA((NBUF,)),
            pltpu.SemaphoreType.DMA((NBUF,)),
        ],
    )(x3, table)


def kernel(x, table):
    x3 = x.reshape(NW, NSTEPS * GPB, C).astype(jnp.int32)
    out = _emb(x3, table)
    return out.reshape(x.shape[0], x.shape[1], D)


# speedup vs baseline: 1.0111x; 1.0111x over previous
import jax
import jax.numpy as jnp
from jax import lax
from jax.experimental import pallas as pl
from jax.experimental.pallas import tpu as pltpu
from jax.experimental.pallas import tpu_sc as plsc

TOKENS = 4096 * 200
D = 128
NC, NS = 2, 16
NW = NC * NS
B_PER_W = TOKENS // NW
C = 128
GPB = 1
ROWS = C * GPB
NSTEPS = B_PER_W // ROWS
NBUF = 4
NSUPER = NSTEPS // NBUF


def _emb_body(x_hbm, table_hbm, out_hbm, idx_v, bufs, gsem, ssem):
    wid = lax.axis_index("s") * NC + lax.axis_index("c")
    base = wid * B_PER_W

    pltpu.sync_copy(x_hbm.at[wid], idx_v)

    def fire_gather(b, c):
        for j in range(GPB):
            pltpu.async_copy(
                table_hbm.at[idx_v.at[c * GPB + j]],
                bufs.at[b, pl.ds(j * C, C)],
                gsem.at[b],
            )

    def wait_gather(b):
        pltpu.make_async_copy(
            table_hbm.at[pl.ds(0, ROWS)], bufs.at[b], gsem.at[b]
        ).wait()

    def fire_store(b, c):
        pltpu.async_copy(
            bufs.at[b], out_hbm.at[pl.ds(base + c * ROWS, ROWS)], ssem.at[b]
        )

    def wait_store(b):
        pltpu.make_async_copy(
            bufs.at[b], out_hbm.at[pl.ds(0, ROWS)], ssem.at[b]
        ).wait()

    for b in range(NBUF):
        fire_gather(b, b)

    def body(s, carry):
        g = s * NBUF
        for b in range(NBUF):
            wait_gather(b)
            fire_store(b, g + b)
        for b in range(NBUF):
            wait_store(b)
            fire_gather(b, g + b + NBUF)
        return carry

    lax.fori_loop(0, NSUPER - 1, body, 0, unroll=False)

    g = (NSUPER - 1) * NBUF
    for b in range(NBUF):
        wait_gather(b)
        fire_store(b, g + b)
    for b in range(NBUF):
        wait_store(b)


@jax.jit
def _emb(x3, table):
    mesh = plsc.VectorSubcoreMesh(
        core_axis_name="c", subcore_axis_name="s", num_cores=NC, num_subcores=NS
    )
    return pl.kernel(
        _emb_body,
        out_type=jax.ShapeDtypeStruct((TOKENS, D), jnp.float32),
        mesh=mesh,
        scratch_types=[
            pltpu.VMEM((NSTEPS * GPB, C), jnp.int32),
            pltpu.VMEM((NBUF, ROWS, D), jnp.float32),
            pltpu.SemaphoreType.DMA((NBUF,)),
            pltpu.SemaphoreType.DMA((NBUF,)),
        ],
    )(x3, table)


def kernel(x, table):
    x3 = x.reshape(NW, NSTEPS * GPB, C).astype(jnp.int32)
    out = _emb(x3, table)
    return out.reshape(x.shape[0], x.shape[1], D)


# speedup vs baseline: 1.0463x; 1.0348x over previous
import jax
import jax.numpy as jnp
from jax import lax
from jax.experimental import pallas as pl
from jax.experimental.pallas import tpu as pltpu
from jax.experimental.pallas import tpu_sc as plsc

TOKENS = 4096 * 200
D = 128
NC, NS = 2, 16
NW = NC * NS
B_PER_W = TOKENS // NW
C = 128
GPB = 1
ROWS = C * GPB
NSTEPS = B_PER_W // ROWS
NBUF = 4
NSUPER = NSTEPS // NBUF


def _emb_body(x_hbm, table_hbm, out_hbm, idx_v, bufs, shared, gsem, csem, ssem):
    sid = lax.axis_index("s")
    wid = lax.axis_index("s") * NC + lax.axis_index("c")
    base = wid * B_PER_W

    pltpu.sync_copy(x_hbm.at[wid], idx_v)

    def fire_gather(b, c):
        for j in range(GPB):
            pltpu.async_copy(
                table_hbm.at[idx_v.at[c * GPB + j]],
                bufs.at[b, pl.ds(j * C, C)],
                gsem.at[b],
            )

    def wait_gather(b):
        pltpu.make_async_copy(
            table_hbm.at[pl.ds(0, ROWS)], bufs.at[b], gsem.at[b]
        ).wait()

    def fire_store(b, c):
        if b % 2 == 0:
            pltpu.async_copy(
                bufs.at[b], out_hbm.at[pl.ds(base + c * ROWS, ROWS)], ssem.at[b]
            )
        else:
            pltpu.async_copy(bufs.at[b], shared.at[sid, b // 2], csem.at[b // 2])
            pltpu.make_async_copy(
                bufs.at[b], shared.at[sid, b // 2], csem.at[b // 2]
            ).wait()
            pltpu.async_copy(
                shared.at[sid, b // 2],
                out_hbm.at[pl.ds(base + c * ROWS, ROWS)],
                ssem.at[b],
            )

    def wait_store(b):
        if b % 2 == 0:
            pltpu.make_async_copy(
                bufs.at[b], out_hbm.at[pl.ds(0, ROWS)], ssem.at[b]
            ).wait()
        else:
            pltpu.make_async_copy(
                shared.at[sid, b // 2], out_hbm.at[pl.ds(0, ROWS)], ssem.at[b]
            ).wait()

    for b in range(NBUF):
        fire_gather(b, b)

    def body(s, carry):
        g = s * NBUF
        for b in range(NBUF):
            wait_gather(b)
            fire_store(b, g + b)
        for b in range(NBUF):
            wait_store(b)
            fire_gather(b, g + b + NBUF)
        return carry

    lax.fori_loop(0, NSUPER - 1, body, 0, unroll=False)

    g = (NSUPER - 1) * NBUF
    for b in range(NBUF):
        wait_gather(b)
        fire_store(b, g + b)
    for b in range(NBUF):
        wait_store(b)


@jax.jit
def _emb(x3, table):
    mesh = plsc.VectorSubcoreMesh(
        core_axis_name="c", subcore_axis_name="s", num_cores=NC, num_subcores=NS
    )
    return pl.kernel(
        _emb_body,
        out_type=jax.ShapeDtypeStruct((TOKENS, D), jnp.float32),
        mesh=mesh,
        scratch_types=[
            pltpu.VMEM((NSTEPS * GPB, C), jnp.int32),
            pltpu.VMEM((NBUF, ROWS, D), jnp.float32),
            pltpu.VMEM_SHARED((NS, NBUF // 2, ROWS, D), jnp.float32),
            pltpu.SemaphoreType.DMA((NBUF // 2,)),
            pltpu.SemaphoreType.DMA((NBUF,)),
            pltpu.SemaphoreType.DMA((NBUF,)),
        ],
    )(x3, table)


def kernel(x, table):
    x3 = x.reshape(NW, NSTEPS * GPB, C).astype(jnp.int32)
    out = _emb(x3, table)
    return out.reshape(x.shape[0], x.shape[1], D)
